# baseline (speedup 1.0000x reference)
import functools

import jax
import jax.numpy as jnp
from jax import lax
from jax.experimental import pallas as pl
from jax.experimental.pallas import tpu as pltpu
from jax.experimental.pallas import tpu_sc as plsc

N_TILES = 16
N_CORES = 2
LANE = 128


def _round_up(x, m):
    return (x + m - 1) // m * m


def _sc_mesh():
    return plsc.VectorSubcoreMesh(core_axis_name="c", subcore_axis_name="s")


def _worker_rows(r):
    q, rem = r // (N_TILES * N_CORES), r % (N_TILES * N_CORES)

    def split(wid):
        start = wid * q + jnp.minimum(wid, rem)
        extra = jnp.where(wid < rem, 1, 0)
        return start, q, extra

    return split


def _make_deg_kernel(n_pad, r, chunk):
    slice_len = n_pad // N_TILES
    split = _worker_rows(r)
    n_chunks = (r // (N_TILES * N_CORES)) // chunk

    @functools.partial(
        pl.kernel,
        out_type=jax.ShapeDtypeStruct((N_CORES, n_pad), jnp.float32),
        mesh=_sc_mesh(),
        scratch_types=[
            pltpu.VMEM_SHARED((n_pad,), jnp.float32),
            pltpu.VMEM((chunk, LANE), jnp.int32),
            pltpu.VMEM((chunk, LANE), jnp.int32),
            pltpu.VMEM((LANE,), jnp.float32),
            pltpu.SemaphoreType.DMA,
            pltpu.SemaphoreType.DMA,
        ],
        compiler_params=pltpu.CompilerParams(use_tc_tiling_on_sc=False),
    )
    def deg_kernel(edge_hbm, zeros_hbm, ones_hbm, out_hbm, deg_sh, dst_buf, dst_buf2,
                   ones_buf, ssem, isem):
        c = lax.axis_index("c")
        s = lax.axis_index("s")
        wid = s * N_CORES + c
        tb = s * slice_len
        dst_hbm = edge_hbm.at[1]
        pltpu.sync_copy(zeros_hbm.at[pl.ds(tb, slice_len)],
                        deg_sh.at[pl.ds(tb, slice_len)])
        pltpu.sync_copy(ones_hbm, ones_buf)
        plsc.subcore_barrier()
        base, q, extra = split(wid)

        def body(i, carry):
            row_a = base + (2 * i) * chunk
            pltpu.sync_copy(dst_hbm.at[pl.ds(row_a, chunk)], dst_buf)
            sa = [pltpu.async_copy(ones_buf, deg_sh.at[dst_buf.at[j]], ssem,
                                   add=True)
                  for j in range(chunk)]
            ib = pltpu.async_copy(dst_hbm.at[pl.ds(row_a + chunk, chunk)],
                                  dst_buf2, isem)
            ib.wait()
            sb = [pltpu.async_copy(ones_buf, deg_sh.at[dst_buf2.at[j]], ssem,
                                   add=True)
                  for j in range(chunk)]
            for d in sa + sb:
                d.wait()
            return carry

        lax.fori_loop(0, n_chunks // 2, body, 0)
        if n_chunks % 2:
            pltpu.sync_copy(
                dst_hbm.at[pl.ds(base + (n_chunks - 1) * chunk, chunk)], dst_buf)
            descs = [pltpu.async_copy(ones_buf, deg_sh.at[dst_buf.at[j]], ssem,
                                      add=True)
                     for j in range(chunk)]
            for d in descs:
                d.wait()

        def tail(i, carry):
            pltpu.sync_copy(dst_hbm.at[pl.ds(base + n_chunks * chunk + i, 1)],
                            dst_buf.at[pl.ds(0, 1)])
            pltpu.sync_copy(ones_buf, deg_sh.at[dst_buf.at[0]], add=True)
            return carry

        lax.fori_loop(0, q - n_chunks * chunk + extra, tail, 0)
        plsc.subcore_barrier()
        pltpu.sync_copy(deg_sh.at[pl.ds(tb, slice_len)],
                        out_hbm.at[c].at[pl.ds(tb, slice_len)])

    return deg_kernel


def _make_agg_kernel(n_pad, r, feat, chunk):
    slice_len = n_pad // N_TILES
    split = _worker_rows(r)
    n_chunks = (r // (N_TILES * N_CORES)) // chunk

    @functools.partial(
        pl.kernel,
        out_type=jax.ShapeDtypeStruct((N_CORES, n_pad, feat), jnp.float32),
        mesh=_sc_mesh(),
        scratch_types=[
            pltpu.VMEM_SHARED((n_pad, feat), jnp.float32),
            pltpu.VMEM_SHARED((n_pad, feat), jnp.float32),
            pltpu.VMEM((chunk, LANE), jnp.int32),
            pltpu.VMEM((chunk, LANE), jnp.int32),
            pltpu.VMEM((chunk, LANE, feat), jnp.float32),
            pltpu.VMEM((chunk, LANE), jnp.int32),
            pltpu.VMEM((chunk, LANE), jnp.int32),
            pltpu.VMEM((chunk, LANE, feat), jnp.float32),
            pltpu.VMEM((chunk, LANE), jnp.int32),
            pltpu.VMEM((chunk, LANE), jnp.int32),
            pltpu.VMEM((chunk, LANE, feat), jnp.float32),
            pltpu.SemaphoreType.DMA,
            pltpu.SemaphoreType.DMA,
            pltpu.SemaphoreType.DMA,
        ],
        compiler_params=pltpu.CompilerParams(use_tc_tiling_on_sc=False),
    )
    def agg_kernel(g_hbm, edge_hbm, zeros_hbm, out_hbm,
                   g_sh, acc_sh, src_buf, dst_buf, rows_buf,
                   src_buf2, dst_buf2, rows_buf2,
                   src_buf3, dst_buf3, rows_buf3, gsem, ssem, isem):
        c = lax.axis_index("c")
        s = lax.axis_index("s")
        wid = s * N_CORES + c
        tb = s * slice_len
        src_hbm = edge_hbm.at[0]
        dst_hbm = edge_hbm.at[1]
        pltpu.sync_copy(zeros_hbm.at[pl.ds(tb, slice_len)],
                        acc_sh.at[pl.ds(tb, slice_len)])
        pltpu.sync_copy(g_hbm.at[pl.ds(tb, slice_len)],
                        g_sh.at[pl.ds(tb, slice_len)])
        plsc.subcore_barrier()
        base, q, extra = split(wid)

        def load_idx(row0, sbuf, dbuf):
            return [pltpu.async_copy(src_hbm.at[pl.ds(row0, chunk)], sbuf, isem),
                    pltpu.async_copy(dst_hbm.at[pl.ds(row0, chunk)], dbuf, isem)]

        def fire_gathers(sbuf, rbuf):
            return [pltpu.async_copy(g_sh.at[sbuf.at[j]], rbuf.at[j], gsem)
                    for j in range(chunk)]

        def fire_scatters(dbuf, rbuf):
            return [pltpu.async_copy(rbuf.at[j], acc_sh.at[dbuf.at[j]], ssem,
                                     add=True)
                    for j in range(chunk)]

        def drain(descs):
            for d in descs:
                d.wait()

        def body(i, carry):
            row_a = base + (3 * i) * chunk
            drain(load_idx(row_a, src_buf, dst_buf))
            ga = fire_gathers(src_buf, rows_buf)
            ib = load_idx(row_a + chunk, src_buf2, dst_buf2)
            drain(ga)
            sa = fire_scatters(dst_buf, rows_buf)
            drain(ib)
            gb = fire_gathers(src_buf2, rows_buf2)
            ic = load_idx(row_a + 2 * chunk, src_buf3, dst_buf3)
            drain(gb)
            sb = fire_scatters(dst_buf2, rows_buf2)
            drain(ic)
            gc = fire_gathers(src_buf3, rows_buf3)
            drain(sa)
            drain(gc)
            sc = fire_scatters(dst_buf3, rows_buf3)
            drain(sb)
            drain(sc)
            return carry

        lax.fori_loop(0, n_chunks // 3, body, 0)
        for k in range(n_chunks % 3):
            row0 = base + (n_chunks - (n_chunks % 3) + k) * chunk
            drain(load_idx(row0, src_buf, dst_buf))
            drain(fire_gathers(src_buf, rows_buf))
            drain(fire_scatters(dst_buf, rows_buf))

        def tail(i, carry):
            rb = base + n_chunks * chunk + i
            pltpu.sync_copy(src_hbm.at[pl.ds(rb, 1)], src_buf.at[pl.ds(0, 1)])
            pltpu.sync_copy(dst_hbm.at[pl.ds(rb, 1)], dst_buf.at[pl.ds(0, 1)])
            pltpu.sync_copy(g_sh.at[src_buf.at[0]], rows_buf.at[0])
            pltpu.sync_copy(rows_buf.at[0], acc_sh.at[dst_buf.at[0]], add=True)
            return carry

        lax.fori_loop(0, q - n_chunks * chunk + extra, tail, 0)
        plsc.subcore_barrier()
        pltpu.sync_copy(acc_sh.at[pl.ds(tb, slice_len)],
                        out_hbm.at[c].at[pl.ds(tb, slice_len)])

    return agg_kernel


def _pad_cols(v, width):
    b, f = v.shape
    if f == width:
        return v
    return jnp.concatenate([v, jnp.zeros((b, width - f), v.dtype)], axis=1)


def _d1_body(degp_ref, x_ref, w1_ref, s_ref, g1_ref):
    deg = 1.0 + degp_ref[0] + degp_ref[1]
    s = lax.rsqrt(deg)
    s_ref[...] = s
    h = jnp.dot(x_ref[...], w1_ref[...], preferred_element_type=jnp.float32)
    g1_ref[...] = _pad_cols(s * h, g1_ref.shape[1])


def _d2_body(e1p_ref, g1_ref, s_ref, w2_ref, b1_ref, g2_ref):
    f1 = w2_ref.shape[0]
    s = s_ref[...]
    e1 = e1p_ref[0, :, :f1] + e1p_ref[1, :, :f1] + g1_ref[:, :f1]
    h1 = jnp.maximum(s * e1 + b1_ref[...], 0.0)
    g2 = s * jnp.dot(h1, w2_ref[...], preferred_element_type=jnp.float32)
    g2_ref[...] = _pad_cols(g2, g2_ref.shape[1])


def _d3_body(e2p_ref, g2_ref, s_ref, b2_ref, out_ref):
    f2 = out_ref.shape[1]
    logits = (s_ref[...] * (e2p_ref[0, :, :f2] + e2p_ref[1, :, :f2] + g2_ref[:, :f2])
              + b2_ref[...])
    m = jnp.max(logits, axis=1, keepdims=True)
    lse = m + jnp.log(jnp.sum(jnp.exp(logits - m), axis=1, keepdims=True))
    out_ref[...] = logits - lse


def _dense_call(body, grid, in_specs, out_specs, out_shape):
    return pl.pallas_call(body, grid=grid, in_specs=in_specs,
                          out_specs=out_specs, out_shape=out_shape)


def kernel(x, edge_index, W1, b1, W2, b2):
    n = x.shape[0]
    e = edge_index.shape[1]
    f1 = W1.shape[1]
    f2 = W2.shape[1]

    block = 6400
    n_pad = _round_up(n, block)
    grid_n = n_pad // block

    chunk = 6
    ei = edge_index.astype(jnp.int32)
    pad_e = _round_up(e, LANE) - e
    if pad_e:
        pad_col = jnp.concatenate([jnp.zeros((1, pad_e), jnp.int32),
                                   jnp.full((1, pad_e), n, jnp.int32)])
        ei = jnp.concatenate([ei, pad_col], axis=1)
    r = (e + pad_e) // LANE
    edge2 = ei.reshape(2, r, LANE)

    x_pad = jnp.pad(x, ((0, n_pad - n), (0, 0)))
    zeros1 = jnp.zeros((n_pad,), jnp.float32)
    ones_l = jnp.ones((LANE,), jnp.float32)

    deg_k = _make_deg_kernel(n_pad, r, 2 * chunk)
    degp = deg_k(edge2, zeros1, ones_l)
    degp3 = degp.reshape(N_CORES, n_pad, 1)

    fp1, fp2 = 8, 8
    s_arr, g1 = _dense_call(
        _d1_body, (grid_n,),
        [pl.BlockSpec((N_CORES, block, 1), lambda i: (0, i, 0)),
         pl.BlockSpec((block, x.shape[1]), lambda i: (i, 0)),
         pl.BlockSpec(W1.shape, lambda i: (0, 0))],
        [pl.BlockSpec((block, 1), lambda i: (i, 0)),
         pl.BlockSpec((block, fp1), lambda i: (i, 0))],
        [jax.ShapeDtypeStruct((n_pad, 1), jnp.float32),
         jax.ShapeDtypeStruct((n_pad, fp1), jnp.float32)],
    )(degp3, x_pad, W1)

    agg1 = _make_agg_kernel(n_pad, r, fp1, chunk)
    e1p = agg1(g1, edge2, jnp.zeros((n_pad, fp1), jnp.float32))

    g2 = _dense_call(
        _d2_body, (grid_n,),
        [pl.BlockSpec((N_CORES, block, fp1), lambda i: (0, i, 0)),
         pl.BlockSpec((block, fp1), lambda i: (i, 0)),
         pl.BlockSpec((block, 1), lambda i: (i, 0)),
         pl.BlockSpec(W2.shape, lambda i: (0, 0)),
         pl.BlockSpec((1, f1), lambda i: (0, 0))],
        pl.BlockSpec((block, fp2), lambda i: (i, 0)),
        jax.ShapeDtypeStruct((n_pad, fp2), jnp.float32),
    )(e1p, g1, s_arr, W2, b1.reshape(1, f1))

    e2p = agg1(g2, edge2, jnp.zeros((n_pad, fp2), jnp.float32))

    out = _dense_call(
        _d3_body, (grid_n,),
        [pl.BlockSpec((N_CORES, block, fp2), lambda i: (0, i, 0)),
         pl.BlockSpec((block, fp2), lambda i: (i, 0)),
         pl.BlockSpec((block, 1), lambda i: (i, 0)),
         pl.BlockSpec((1, f2), lambda i: (0, 0))],
        pl.BlockSpec((block, f2), lambda i: (i, 0)),
        jax.ShapeDtypeStruct((n_pad, f2), jnp.float32),
    )(e2p, g2, s_arr, b2.reshape(1, f2))

    return out[:n]


# speedup vs baseline: 1.0622x; 1.0622x over previous
import functools

import jax
import jax.numpy as jnp
from jax import lax
from jax.experimental import pallas as pl
from jax.experimental.pallas import tpu as pltpu
from jax.experimental.pallas import tpu_sc as plsc

N_TILES = 16
N_CORES = 2
LANE = 128


def _round_up(x, m):
    return (x + m - 1) // m * m


def _sc_mesh():
    return plsc.VectorSubcoreMesh(core_axis_name="c", subcore_axis_name="s")


def _worker_rows(r):
    q, rem = r // (N_TILES * N_CORES), r % (N_TILES * N_CORES)

    def split(wid):
        start = wid * q + jnp.minimum(wid, rem)
        extra = jnp.where(wid < rem, 1, 0)
        return start, q, extra

    return split


def _make_deg_kernel(n_pad, r, chunk):
    slice_len = n_pad // N_TILES
    split = _worker_rows(r)
    n_chunks = (r // (N_TILES * N_CORES)) // chunk

    @functools.partial(
        pl.kernel,
        out_type=jax.ShapeDtypeStruct((N_CORES, n_pad), jnp.float32),
        mesh=_sc_mesh(),
        scratch_types=[
            pltpu.VMEM_SHARED((n_pad,), jnp.float32),
            pltpu.VMEM((chunk, LANE), jnp.int32),
            pltpu.VMEM((chunk, LANE), jnp.int32),
            pltpu.VMEM((LANE,), jnp.float32),
            pltpu.SemaphoreType.DMA,
            pltpu.SemaphoreType.DMA,
        ],
        compiler_params=pltpu.CompilerParams(use_tc_tiling_on_sc=False),
    )
    def deg_kernel(edge_hbm, zeros_hbm, ones_hbm, out_hbm, deg_sh, dst_buf, dst_buf2,
                   ones_buf, ssem, isem):
        c = lax.axis_index("c")
        s = lax.axis_index("s")
        wid = s * N_CORES + c
        tb = s * slice_len
        dst_hbm = edge_hbm.at[1]
        pltpu.sync_copy(zeros_hbm.at[pl.ds(tb, slice_len)],
                        deg_sh.at[pl.ds(tb, slice_len)])
        pltpu.sync_copy(ones_hbm, ones_buf)
        plsc.subcore_barrier()
        base, q, extra = split(wid)

        def body(i, carry):
            row_a = base + (2 * i) * chunk
            pltpu.sync_copy(dst_hbm.at[pl.ds(row_a, chunk)], dst_buf)
            sa = [pltpu.async_copy(ones_buf, deg_sh.at[dst_buf.at[j]], ssem,
                                   add=True)
                  for j in range(chunk)]
            ib = pltpu.async_copy(dst_hbm.at[pl.ds(row_a + chunk, chunk)],
                                  dst_buf2, isem)
            ib.wait()
            sb = [pltpu.async_copy(ones_buf, deg_sh.at[dst_buf2.at[j]], ssem,
                                   add=True)
                  for j in range(chunk)]
            for d in sa + sb:
                d.wait()
            return carry

        lax.fori_loop(0, n_chunks // 2, body, 0)
        if n_chunks % 2:
            pltpu.sync_copy(
                dst_hbm.at[pl.ds(base + (n_chunks - 1) * chunk, chunk)], dst_buf)
            descs = [pltpu.async_copy(ones_buf, deg_sh.at[dst_buf.at[j]], ssem,
                                      add=True)
                     for j in range(chunk)]
            for d in descs:
                d.wait()

        def tail(i, carry):
            pltpu.sync_copy(dst_hbm.at[pl.ds(base + n_chunks * chunk + i, 1)],
                            dst_buf.at[pl.ds(0, 1)])
            pltpu.sync_copy(ones_buf, deg_sh.at[dst_buf.at[0]], add=True)
            return carry

        lax.fori_loop(0, q - n_chunks * chunk + extra, tail, 0)
        plsc.subcore_barrier()
        pltpu.sync_copy(deg_sh.at[pl.ds(tb, slice_len)],
                        out_hbm.at[c].at[pl.ds(tb, slice_len)])

    return deg_kernel


def _make_agg_kernel(n_pad, r, feat, chunk):
    slice_len = n_pad // N_TILES
    split = _worker_rows(r)
    n_chunks = (r // (N_TILES * N_CORES)) // chunk

    @functools.partial(
        pl.kernel,
        out_type=jax.ShapeDtypeStruct((N_CORES, n_pad, feat), jnp.float32),
        mesh=_sc_mesh(),
        scratch_types=[
            pltpu.VMEM_SHARED((n_pad, feat), jnp.float32),
            pltpu.VMEM_SHARED((n_pad, feat), jnp.float32),
            pltpu.VMEM((chunk, LANE), jnp.int32),
            pltpu.VMEM((chunk, LANE), jnp.int32),
            pltpu.VMEM((chunk, LANE, feat), jnp.float32),
            pltpu.VMEM((chunk, LANE), jnp.int32),
            pltpu.VMEM((chunk, LANE), jnp.int32),
            pltpu.VMEM((chunk, LANE, feat), jnp.float32),
            pltpu.SemaphoreType.DMA,
            pltpu.SemaphoreType.DMA,
            pltpu.SemaphoreType.DMA,
        ],
        compiler_params=pltpu.CompilerParams(use_tc_tiling_on_sc=False),
    )
    def agg_kernel(g_hbm, edge_hbm, zeros_hbm, out_hbm,
                   g_sh, acc_sh, src_buf, dst_buf, rows_buf,
                   src_buf2, dst_buf2, rows_buf2, gsem, ssem, isem):
        c = lax.axis_index("c")
        s = lax.axis_index("s")
        wid = s * N_CORES + c
        tb = s * slice_len
        src_hbm = edge_hbm.at[0]
        dst_hbm = edge_hbm.at[1]
        pltpu.sync_copy(zeros_hbm.at[pl.ds(tb, slice_len)],
                        acc_sh.at[pl.ds(tb, slice_len)])
        pltpu.sync_copy(g_hbm.at[pl.ds(tb, slice_len)],
                        g_sh.at[pl.ds(tb, slice_len)])
        plsc.subcore_barrier()
        base, q, extra = split(wid)

        def load_idx(row0, sbuf, dbuf):
            return [pltpu.async_copy(src_hbm.at[pl.ds(row0, chunk)], sbuf, isem),
                    pltpu.async_copy(dst_hbm.at[pl.ds(row0, chunk)], dbuf, isem)]

        def fire_gathers(sbuf, rbuf):
            return [pltpu.async_copy(g_sh.at[sbuf.at[j]], rbuf.at[j], gsem)
                    for j in range(chunk)]

        def fire_scatters(dbuf, rbuf):
            return [pltpu.async_copy(rbuf.at[j], acc_sh.at[dbuf.at[j]], ssem,
                                     add=True)
                    for j in range(chunk)]

        def drain(descs):
            for d in descs:
                d.wait()

        def one_pair(row_a):
            drain(load_idx(row_a, src_buf, dst_buf))
            ga = fire_gathers(src_buf, rows_buf)
            ib = load_idx(row_a + chunk, src_buf2, dst_buf2)
            drain(ga)
            sa = fire_scatters(dst_buf, rows_buf)
            drain(ib)
            drain(fire_gathers(src_buf2, rows_buf2))
            sb = fire_scatters(dst_buf2, rows_buf2)
            drain(sa)
            drain(sb)

        def body(i, carry):
            one_pair(base + (2 * i) * chunk)
            return carry

        lax.fori_loop(0, n_chunks // 2, body, 0)
        if n_chunks % 2:
            row0 = base + (n_chunks - 1) * chunk
            drain(load_idx(row0, src_buf, dst_buf))
            drain(fire_gathers(src_buf, rows_buf))
            drain(fire_scatters(dst_buf, rows_buf))

        def tail(i, carry):
            rb = base + n_chunks * chunk + i
            pltpu.sync_copy(src_hbm.at[pl.ds(rb, 1)], src_buf.at[pl.ds(0, 1)])
            pltpu.sync_copy(dst_hbm.at[pl.ds(rb, 1)], dst_buf.at[pl.ds(0, 1)])
            pltpu.sync_copy(g_sh.at[src_buf.at[0]], rows_buf.at[0])
            pltpu.sync_copy(rows_buf.at[0], acc_sh.at[dst_buf.at[0]], add=True)
            return carry

        lax.fori_loop(0, q - n_chunks * chunk + extra, tail, 0)
        plsc.subcore_barrier()
        pltpu.sync_copy(acc_sh.at[pl.ds(tb, slice_len)],
                        out_hbm.at[c].at[pl.ds(tb, slice_len)])

    return agg_kernel


def _pad_cols(v, width):
    b, f = v.shape
    if f == width:
        return v
    return jnp.concatenate([v, jnp.zeros((b, width - f), v.dtype)], axis=1)


def _d1_body(degp_ref, x_ref, w1_ref, s_ref, g1_ref):
    deg = 1.0 + degp_ref[0] + degp_ref[1]
    s = lax.rsqrt(deg)
    s_ref[...] = s
    h = jnp.dot(x_ref[...], w1_ref[...], preferred_element_type=jnp.float32)
    g1_ref[...] = _pad_cols(s * h, g1_ref.shape[1])


def _d2_body(e1p_ref, g1_ref, s_ref, w2_ref, b1_ref, g2_ref):
    f1 = w2_ref.shape[0]
    s = s_ref[...]
    e1 = e1p_ref[0, :, :f1] + e1p_ref[1, :, :f1] + g1_ref[:, :f1]
    h1 = jnp.maximum(s * e1 + b1_ref[...], 0.0)
    g2 = s * jnp.dot(h1, w2_ref[...], preferred_element_type=jnp.float32)
    g2_ref[...] = _pad_cols(g2, g2_ref.shape[1])


def _d3_body(e2p_ref, g2_ref, s_ref, b2_ref, out_ref):
    f2 = out_ref.shape[1]
    logits = (s_ref[...] * (e2p_ref[0, :, :f2] + e2p_ref[1, :, :f2] + g2_ref[:, :f2])
              + b2_ref[...])
    m = jnp.max(logits, axis=1, keepdims=True)
    lse = m + jnp.log(jnp.sum(jnp.exp(logits - m), axis=1, keepdims=True))
    out_ref[...] = logits - lse


def _dense_call(body, grid, in_specs, out_specs, out_shape):
    return pl.pallas_call(body, grid=grid, in_specs=in_specs,
                          out_specs=out_specs, out_shape=out_shape)


def kernel(x, edge_index, W1, b1, W2, b2):
    n = x.shape[0]
    e = edge_index.shape[1]
    f1 = W1.shape[1]
    f2 = W2.shape[1]

    block = 6400
    n_pad = _round_up(n, block)
    grid_n = n_pad // block

    chunk = 10
    ei = edge_index.astype(jnp.int32)
    pad_e = _round_up(e, LANE) - e
    if pad_e:
        pad_col = jnp.concatenate([jnp.zeros((1, pad_e), jnp.int32),
                                   jnp.full((1, pad_e), n, jnp.int32)])
        ei = jnp.concatenate([ei, pad_col], axis=1)
    r = (e + pad_e) // LANE
    edge2 = ei.reshape(2, r, LANE)

    x_pad = jnp.pad(x, ((0, n_pad - n), (0, 0)))
    zeros1 = jnp.zeros((n_pad,), jnp.float32)
    ones_l = jnp.ones((LANE,), jnp.float32)

    deg_k = _make_deg_kernel(n_pad, r, 2 * chunk)
    degp = deg_k(edge2, zeros1, ones_l)
    degp3 = degp.reshape(N_CORES, n_pad, 1)

    fp1, fp2 = 8, 8
    s_arr, g1 = _dense_call(
        _d1_body, (grid_n,),
        [pl.BlockSpec((N_CORES, block, 1), lambda i: (0, i, 0)),
         pl.BlockSpec((block, x.shape[1]), lambda i: (i, 0)),
         pl.BlockSpec(W1.shape, lambda i: (0, 0))],
        [pl.BlockSpec((block, 1), lambda i: (i, 0)),
         pl.BlockSpec((block, fp1), lambda i: (i, 0))],
        [jax.ShapeDtypeStruct((n_pad, 1), jnp.float32),
         jax.ShapeDtypeStruct((n_pad, fp1), jnp.float32)],
    )(degp3, x_pad, W1)

    agg1 = _make_agg_kernel(n_pad, r, fp1, chunk)
    e1p = agg1(g1, edge2, jnp.zeros((n_pad, fp1), jnp.float32))

    g2 = _dense_call(
        _d2_body, (grid_n,),
        [pl.BlockSpec((N_CORES, block, fp1), lambda i: (0, i, 0)),
         pl.BlockSpec((block, fp1), lambda i: (i, 0)),
         pl.BlockSpec((block, 1), lambda i: (i, 0)),
         pl.BlockSpec(W2.shape, lambda i: (0, 0)),
         pl.BlockSpec((1, f1), lambda i: (0, 0))],
        pl.BlockSpec((block, fp2), lambda i: (i, 0)),
        jax.ShapeDtypeStruct((n_pad, fp2), jnp.float32),
    )(e1p, g1, s_arr, W2, b1.reshape(1, f1))

    e2p = agg1(g2, edge2, jnp.zeros((n_pad, fp2), jnp.float32))

    out = _dense_call(
        _d3_body, (grid_n,),
        [pl.BlockSpec((N_CORES, block, fp2), lambda i: (0, i, 0)),
         pl.BlockSpec((block, fp2), lambda i: (i, 0)),
         pl.BlockSpec((block, 1), lambda i: (i, 0)),
         pl.BlockSpec((1, f2), lambda i: (0, 0))],
        pl.BlockSpec((block, f2), lambda i: (i, 0)),
        jax.ShapeDtypeStruct((n_pad, f2), jnp.float32),
    )(e2p, g2, s_arr, b2.reshape(1, f2))

    return out[:n]


# speedup vs baseline: 1.0804x; 1.0171x over previous
import functools

import jax
import jax.numpy as jnp
from jax import lax
from jax.experimental import pallas as pl
from jax.experimental.pallas import tpu as pltpu
from jax.experimental.pallas import tpu_sc as plsc

N_TILES = 16
N_CORES = 2
LANE = 128


def _round_up(x, m):
    return (x + m - 1) // m * m


def _sc_mesh():
    return plsc.VectorSubcoreMesh(core_axis_name="c", subcore_axis_name="s")


def _worker_rows(r):
    q, rem = r // (N_TILES * N_CORES), r % (N_TILES * N_CORES)

    def split(wid):
        start = wid * q + jnp.minimum(wid, rem)
        extra = jnp.where(wid < rem, 1, 0)
        return start, q, extra

    return split


def _make_deg_kernel(n_pad, r, chunk):
    slice_len = n_pad // N_TILES
    split = _worker_rows(r)
    n_chunks = (r // (N_TILES * N_CORES)) // chunk

    @functools.partial(
        pl.kernel,
        out_type=jax.ShapeDtypeStruct((N_CORES, n_pad), jnp.float32),
        mesh=_sc_mesh(),
        scratch_types=[
            pltpu.VMEM_SHARED((n_pad,), jnp.float32),
            pltpu.VMEM((chunk, LANE), jnp.int32),
            pltpu.VMEM((chunk, LANE), jnp.int32),
            pltpu.VMEM((LANE,), jnp.float32),
            pltpu.SemaphoreType.DMA,
            pltpu.SemaphoreType.DMA,
        ],
        compiler_params=pltpu.CompilerParams(use_tc_tiling_on_sc=False),
    )
    def deg_kernel(edge_hbm, zeros_hbm, ones_hbm, out_hbm, deg_sh, dst_buf, dst_buf2,
                   ones_buf, ssem, isem):
        c = lax.axis_index("c")
        s = lax.axis_index("s")
        wid = s * N_CORES + c
        tb = s * slice_len
        dst_hbm = edge_hbm.at[1]
        pltpu.sync_copy(zeros_hbm.at[pl.ds(tb, slice_len)],
                        deg_sh.at[pl.ds(tb, slice_len)])
        pltpu.sync_copy(ones_hbm, ones_buf)
        plsc.subcore_barrier()
        base, q, extra = split(wid)

        def body(i, carry):
            row_a = base + (2 * i) * chunk
            pltpu.sync_copy(dst_hbm.at[pl.ds(row_a, chunk)], dst_buf)
            sa = [pltpu.async_copy(ones_buf, deg_sh.at[dst_buf.at[j]], ssem,
                                   add=True)
                  for j in range(chunk)]
            ib = pltpu.async_copy(dst_hbm.at[pl.ds(row_a + chunk, chunk)],
                                  dst_buf2, isem)
            ib.wait()
            sb = [pltpu.async_copy(ones_buf, deg_sh.at[dst_buf2.at[j]], ssem,
                                   add=True)
                  for j in range(chunk)]
            for d in sa + sb:
                d.wait()
            return carry

        lax.fori_loop(0, n_chunks // 2, body, 0)
        if n_chunks % 2:
            pltpu.sync_copy(
                dst_hbm.at[pl.ds(base + (n_chunks - 1) * chunk, chunk)], dst_buf)
            descs = [pltpu.async_copy(ones_buf, deg_sh.at[dst_buf.at[j]], ssem,
                                      add=True)
                     for j in range(chunk)]
            for d in descs:
                d.wait()

        def tail(i, carry):
            pltpu.sync_copy(dst_hbm.at[pl.ds(base + n_chunks * chunk + i, 1)],
                            dst_buf.at[pl.ds(0, 1)])
            pltpu.sync_copy(ones_buf, deg_sh.at[dst_buf.at[0]], add=True)
            return carry

        lax.fori_loop(0, q - n_chunks * chunk + extra, tail, 0)
        plsc.subcore_barrier()
        pltpu.sync_copy(deg_sh.at[pl.ds(tb, slice_len)],
                        out_hbm.at[c].at[pl.ds(tb, slice_len)])

    return deg_kernel


def _make_agg_kernel(n_pad, r, feat, chunk):
    slice_len = n_pad // N_TILES
    split = _worker_rows(r)
    n_chunks = (r // (N_TILES * N_CORES)) // chunk

    @functools.partial(
        pl.kernel,
        out_type=jax.ShapeDtypeStruct((N_CORES, n_pad, feat), jnp.float32),
        mesh=_sc_mesh(),
        scratch_types=[
            pltpu.VMEM_SHARED((n_pad, feat), jnp.float32),
            pltpu.VMEM_SHARED((n_pad, feat), jnp.float32),
            pltpu.VMEM((chunk, LANE), jnp.int32),
            pltpu.VMEM((chunk, LANE), jnp.int32),
            pltpu.VMEM((chunk, LANE, feat), jnp.float32),
            pltpu.VMEM((chunk, LANE), jnp.int32),
            pltpu.VMEM((chunk, LANE), jnp.int32),
            pltpu.VMEM((chunk, LANE, feat), jnp.float32),
            pltpu.SemaphoreType.DMA,
            pltpu.SemaphoreType.DMA,
            pltpu.SemaphoreType.DMA,
        ],
        compiler_params=pltpu.CompilerParams(use_tc_tiling_on_sc=False),
    )
    def agg_kernel(g_hbm, edge_hbm, zeros_hbm, out_hbm,
                   g_sh, acc_sh, src_buf, dst_buf, rows_buf,
                   src_buf2, dst_buf2, rows_buf2, gsem, ssem, isem):
        c = lax.axis_index("c")
        s = lax.axis_index("s")
        wid = s * N_CORES + c
        tb = s * slice_len
        src_hbm = edge_hbm.at[0]
        dst_hbm = edge_hbm.at[1]
        pltpu.sync_copy(zeros_hbm.at[pl.ds(tb, slice_len)],
                        acc_sh.at[pl.ds(tb, slice_len)])
        pltpu.sync_copy(g_hbm.at[pl.ds(tb, slice_len)],
                        g_sh.at[pl.ds(tb, slice_len)])
        plsc.subcore_barrier()
        base, q, extra = split(wid)

        def load_idx(row0, sbuf, dbuf):
            return [pltpu.async_copy(src_hbm.at[pl.ds(row0, chunk)], sbuf, isem),
                    pltpu.async_copy(dst_hbm.at[pl.ds(row0, chunk)], dbuf, isem)]

        def fire_gathers(sbuf, rbuf):
            return [pltpu.async_copy(g_sh.at[sbuf.at[j]], rbuf.at[j], gsem)
                    for j in range(chunk)]

        def fire_scatters(dbuf, rbuf):
            return [pltpu.async_copy(rbuf.at[j], acc_sh.at[dbuf.at[j]], ssem,
                                     add=True)
                    for j in range(chunk)]

        def drain(descs):
            for d in descs:
                d.wait()

        def one_pair(row_a):
            drain(load_idx(row_a, src_buf, dst_buf))
            ga = fire_gathers(src_buf, rows_buf)
            ib = load_idx(row_a + chunk, src_buf2, dst_buf2)
            drain(ga)
            sa = fire_scatters(dst_buf, rows_buf)
            drain(ib)
            drain(fire_gathers(src_buf2, rows_buf2))
            sb = fire_scatters(dst_buf2, rows_buf2)
            drain(sa)
            drain(sb)

        def body(i, carry):
            one_pair(base + (2 * i) * chunk)
            return carry

        lax.fori_loop(0, n_chunks // 2, body, 0)
        if n_chunks % 2:
            row0 = base + (n_chunks - 1) * chunk
            drain(load_idx(row0, src_buf, dst_buf))
            drain(fire_gathers(src_buf, rows_buf))
            drain(fire_scatters(dst_buf, rows_buf))

        def tail(i, carry):
            rb = base + n_chunks * chunk + i
            pltpu.sync_copy(src_hbm.at[pl.ds(rb, 1)], src_buf.at[pl.ds(0, 1)])
            pltpu.sync_copy(dst_hbm.at[pl.ds(rb, 1)], dst_buf.at[pl.ds(0, 1)])
            pltpu.sync_copy(g_sh.at[src_buf.at[0]], rows_buf.at[0])
            pltpu.sync_copy(rows_buf.at[0], acc_sh.at[dst_buf.at[0]], add=True)
            return carry

        lax.fori_loop(0, q - n_chunks * chunk + extra, tail, 0)
        plsc.subcore_barrier()
        pltpu.sync_copy(acc_sh.at[pl.ds(tb, slice_len)],
                        out_hbm.at[c].at[pl.ds(tb, slice_len)])

    return agg_kernel


def _pad_cols(v, width):
    b, f = v.shape
    if f == width:
        return v
    return jnp.concatenate([v, jnp.zeros((b, width - f), v.dtype)], axis=1)


def _d1_body(degp_ref, x_ref, w1_ref, s_ref, g1_ref):
    deg = 1.0 + degp_ref[0] + degp_ref[1]
    s = lax.rsqrt(deg)
    s_ref[...] = s
    h = jnp.dot(x_ref[...], w1_ref[...], preferred_element_type=jnp.float32)
    g1_ref[...] = _pad_cols(s * h, g1_ref.shape[1])


def _d2_body(e1p_ref, g1_ref, s_ref, w2_ref, b1_ref, g2_ref):
    f1 = w2_ref.shape[0]
    s = s_ref[...]
    e1 = e1p_ref[0, :, :f1] + e1p_ref[1, :, :f1] + g1_ref[:, :f1]
    h1 = jnp.maximum(s * e1 + b1_ref[...], 0.0)
    g2 = s * jnp.dot(h1, w2_ref[...], preferred_element_type=jnp.float32)
    g2_ref[...] = _pad_cols(g2, g2_ref.shape[1])


def _d3_body(e2p_ref, g2_ref, s_ref, b2_ref, out_ref):
    f2 = out_ref.shape[1]
    logits = (s_ref[...] * (e2p_ref[0, :, :f2] + e2p_ref[1, :, :f2] + g2_ref[:, :f2])
              + b2_ref[...])
    m = jnp.max(logits, axis=1, keepdims=True)
    lse = m + jnp.log(jnp.sum(jnp.exp(logits - m), axis=1, keepdims=True))
    out_ref[...] = logits - lse


def _dense_call(body, grid, in_specs, out_specs, out_shape):
    return pl.pallas_call(body, grid=grid, in_specs=in_specs,
                          out_specs=out_specs, out_shape=out_shape)


def kernel(x, edge_index, W1, b1, W2, b2):
    n = x.shape[0]
    e = edge_index.shape[1]
    f1 = W1.shape[1]
    f2 = W2.shape[1]

    block = 6400
    n_pad = _round_up(n, block)
    grid_n = n_pad // block

    chunk = 11
    ei = edge_index.astype(jnp.int32)
    pad_e = _round_up(e, LANE) - e
    if pad_e:
        pad_col = jnp.concatenate([jnp.zeros((1, pad_e), jnp.int32),
                                   jnp.full((1, pad_e), n, jnp.int32)])
        ei = jnp.concatenate([ei, pad_col], axis=1)
    r = (e + pad_e) // LANE
    edge2 = ei.reshape(2, r, LANE)

    x_pad = jnp.pad(x, ((0, n_pad - n), (0, 0)))
    zeros1 = jnp.zeros((n_pad,), jnp.float32)
    ones_l = jnp.ones((LANE,), jnp.float32)

    deg_k = _make_deg_kernel(n_pad, r, 2 * chunk)
    degp = deg_k(edge2, zeros1, ones_l)
    degp3 = degp.reshape(N_CORES, n_pad, 1)

    fp1, fp2 = 8, 8
    s_arr, g1 = _dense_call(
        _d1_body, (grid_n,),
        [pl.BlockSpec((N_CORES, block, 1), lambda i: (0, i, 0)),
         pl.BlockSpec((block, x.shape[1]), lambda i: (i, 0)),
         pl.BlockSpec(W1.shape, lambda i: (0, 0))],
        [pl.BlockSpec((block, 1), lambda i: (i, 0)),
         pl.BlockSpec((block, fp1), lambda i: (i, 0))],
        [jax.ShapeDtypeStruct((n_pad, 1), jnp.float32),
         jax.ShapeDtypeStruct((n_pad, fp1), jnp.float32)],
    )(degp3, x_pad, W1)

    agg1 = _make_agg_kernel(n_pad, r, fp1, chunk)
    e1p = agg1(g1, edge2, jnp.zeros((n_pad, fp1), jnp.float32))

    g2 = _dense_call(
        _d2_body, (grid_n,),
        [pl.BlockSpec((N_CORES, block, fp1), lambda i: (0, i, 0)),
         pl.BlockSpec((block, fp1), lambda i: (i, 0)),
         pl.BlockSpec((block, 1), lambda i: (i, 0)),
         pl.BlockSpec(W2.shape, lambda i: (0, 0)),
         pl.BlockSpec((1, f1), lambda i: (0, 0))],
        pl.BlockSpec((block, fp2), lambda i: (i, 0)),
        jax.ShapeDtypeStruct((n_pad, fp2), jnp.float32),
    )(e1p, g1, s_arr, W2, b1.reshape(1, f1))

    e2p = agg1(g2, edge2, jnp.zeros((n_pad, fp2), jnp.float32))

    out = _dense_call(
        _d3_body, (grid_n,),
        [pl.BlockSpec((N_CORES, block, fp2), lambda i: (0, i, 0)),
         pl.BlockSpec((block, fp2), lambda i: (i, 0)),
         pl.BlockSpec((block, 1), lambda i: (i, 0)),
         pl.BlockSpec((1, f2), lambda i: (0, 0))],
        pl.BlockSpec((block, f2), lambda i: (i, 0)),
        jax.ShapeDtypeStruct((n_pad, f2), jnp.float32),
    )(e2p, g2, s_arr, b2.reshape(1, f2))

    return out[:n]
